# baseline (speedup 1.0000x reference)
import jax
import jax.numpy as jnp
from jax import lax
from jax.experimental import pallas as pl
from jax.experimental.pallas import tpu as pltpu
from jax.experimental.pallas import tpu_sc as plsc

N_ROWS = 10000
D = 128
E = 320000

NUM_CORES = 2
NUM_SUBCORES = 16
CHUNK = 128
NCH = 160
E_PAD = NUM_SUBCORES * CHUNK * NCH
ROWS_PER_TILE = N_ROWS // NUM_SUBCORES
COL, ROW, VAL = 0, 1, 2
WPR = D // 2
NB = 4


def _spmm_body(emb_hbm, recs_hbm, zeros_hbm, out_hbm,
               ibuf0, ibuf1, ibuf2, ibuf3, gbuf0, gbuf1, gbuf2, gbuf3, sbuf,
               accum,
               isem0, isem1, isem2, isem3, gsem0, gsem1, gsem2, gsem3):
    c = lax.axis_index("c")
    s = lax.axis_index("s")
    ibufs = (ibuf0, ibuf1, ibuf2, ibuf3)
    gbufs = (gbuf0, gbuf1, gbuf2, gbuf3)
    isems = (isem0, isem1, isem2, isem3)
    gsems = (gsem0, gsem1, gsem2, gsem3)
    himask = jnp.full((16,), -65536, dtype=jnp.int32)

    pltpu.sync_copy(zeros_hbm, accum.at[pl.ds(s * ROWS_PER_TILE, ROWS_PER_TILE)])

    for k in range(NB):
        pltpu.async_copy(recs_hbm.at[c, s, k], ibufs[k], isems[k])
    plsc.subcore_barrier()
    for k in range(NB - 1):
        pltpu.make_async_copy(recs_hbm.at[c, s, 0], ibufs[k], isems[k]).wait()
        pltpu.async_copy(emb_hbm.at[ibufs[k].at[COL]], gbufs[k], gsems[k])

    def quad_body(mm, carry):
        for k in range(NB):
            m = mm * NB + k
            k3 = (k + 3) % NB

            @pl.when(mm * NB + k + 3 < NCH)
            def _(_k3=k3):
                pltpu.make_async_copy(
                    recs_hbm.at[c, s, 0], ibufs[_k3], isems[_k3]).wait()
                pltpu.async_copy(
                    emb_hbm.at[ibufs[_k3].at[COL]], gbufs[_k3], gsems[_k3])

            pltpu.make_async_copy(
                emb_hbm.at[ibufs[k].at[COL]], gbufs[k], gsems[k]).wait()

            @plsc.parallel_loop(0, CHUNK // 16, unroll=1)
            def group_body(g, _k=k):
                vvec = lax.bitcast_convert_type(
                    ibufs[_k][VAL, pl.ds(g * 16, 16)], jnp.float32)
                for lane in range(16):
                    vv = jnp.full((16,), vvec[lane], dtype=jnp.float32)
                    e = g * 16 + lane
                    for d in range(WPR // 16):
                        w = gbufs[_k][e, pl.ds(d * 16, 16)]
                        lo = lax.bitcast_convert_type(
                            lax.shift_left(w, 16), jnp.float32)
                        hi = lax.bitcast_convert_type(w & himask, jnp.float32)
                        sbuf[e, pl.ds(d * 32, 16)] = lo * vv
                        sbuf[e, pl.ds(d * 32 + 16, 16)] = hi * vv

            pltpu.sync_copy(sbuf, accum.at[ibufs[k].at[ROW]], add=True)

            @pl.when(mm * NB + k + NB < NCH)
            def _(_k=k, _m=m):
                pltpu.async_copy(recs_hbm.at[c, s, _m + NB], ibufs[_k], isems[_k])
        return carry

    lax.fori_loop(0, NCH // NB, quad_body, 0, unroll=False)

    plsc.subcore_barrier()

    pltpu.sync_copy(
        accum.at[pl.ds(s * ROWS_PER_TILE, ROWS_PER_TILE)],
        out_hbm.at[c, s],
    )


@jax.jit
def kernel(users_emb, items_emb, user_indices, user_values, item_indices, item_values):
    emb = jnp.concatenate([users_emb, items_emb], axis=0)
    emb = lax.bitcast_convert_type(
        emb.astype(jnp.bfloat16).reshape(2 * N_ROWS, WPR, 2), jnp.int32)

    def prep(a):
        a = a.astype(jnp.int32)
        a = jnp.concatenate([a, jnp.zeros((E_PAD - E,), jnp.int32)])
        return a.reshape(NUM_SUBCORES, NCH, 1, CHUNK)

    recs = jnp.stack([
        jnp.concatenate([
            prep(user_indices[1]),
            prep(user_indices[0]),
            prep(lax.bitcast_convert_type(user_values, jnp.int32)),
        ], axis=2),
        jnp.concatenate([
            prep(item_indices[1] + N_ROWS),
            prep(item_indices[0]),
            prep(lax.bitcast_convert_type(item_values, jnp.int32)),
        ], axis=2),
    ])
    zeros = jnp.zeros((ROWS_PER_TILE, D), jnp.float32)

    mesh = plsc.VectorSubcoreMesh(
        core_axis_name="c", subcore_axis_name="s",
        num_cores=NUM_CORES, num_subcores=NUM_SUBCORES,
    )
    out = pl.kernel(
        _spmm_body,
        out_type=jax.ShapeDtypeStruct(
            (NUM_CORES, NUM_SUBCORES, ROWS_PER_TILE, D), jnp.float32),
        mesh=mesh,
        compiler_params=pltpu.CompilerParams(use_tc_tiling_on_sc=False),
        scratch_types=(
            [pltpu.VMEM((3, CHUNK), jnp.int32) for _ in range(NB)]
            + [pltpu.VMEM((CHUNK, WPR), jnp.int32) for _ in range(NB)]
            + [pltpu.VMEM((CHUNK, D), jnp.float32)]
            + [pltpu.VMEM_SHARED((N_ROWS, D), jnp.float32)]
            + [pltpu.SemaphoreType.DMA] * (2 * NB)
        ),
    )(emb, recs, zeros)

    out = out.reshape(NUM_CORES, N_ROWS, D // 32, 2, 16)
    out = out.transpose(0, 1, 2, 4, 3).reshape(NUM_CORES, N_ROWS, D)
    return (out[0], out[1])


# speedup vs baseline: 1.3312x; 1.3312x over previous
import jax
import jax.numpy as jnp
from jax import lax
from jax.experimental import pallas as pl
from jax.experimental.pallas import tpu as pltpu
from jax.experimental.pallas import tpu_sc as plsc

N_ROWS = 10000
D = 128
E = 320000

NUM_CORES = 2
NUM_SUBCORES = 16
CHUNK = 128
NCH = 158
E_PAD = NUM_SUBCORES * CHUNK * NCH
ROWS_PER_TILE = N_ROWS // NUM_SUBCORES
CR, VAL = 0, 1


def _spmm_body(emb_hbm, recs_hbm, zeros_hbm, out_hbm,
               ibuf0, ibuf1, gbuf0, gbuf1, sbuf, colbuf, rowbuf, accum,
               isem0, isem1, gsem0, gsem1):
    c = lax.axis_index("c")
    s = lax.axis_index("s")
    ibufs = (ibuf0, ibuf1)
    gbufs = (gbuf0, gbuf1)
    isems = (isem0, isem1)
    gsems = (gsem0, gsem1)
    lomask = jnp.full((16,), 65535, dtype=jnp.int32)

    pltpu.sync_copy(zeros_hbm, accum.at[pl.ds(s * ROWS_PER_TILE, ROWS_PER_TILE)])

    for b in range(2):
        pltpu.async_copy(recs_hbm.at[c, s, b], ibufs[b], isems[b])
    plsc.subcore_barrier()
    pltpu.make_async_copy(recs_hbm.at[c, s, 0], ibuf0, isem0).wait()
    for q in range(CHUNK // 16):
        colbuf[0, pl.ds(q * 16, 16)] = ibuf0[CR, pl.ds(q * 16, 16)] & lomask
    pltpu.async_copy(emb_hbm.at[colbuf.at[0]], gbuf0, gsem0)

    npair = NCH // 2

    def pair_body(jj, carry):
        for b in range(2):
            j = jj * 2 + b
            o = 1 - b

            @pl.when(jj * 2 + b + 1 < NCH)
            def _(_b=b, _o=o, _j=j):
                pltpu.make_async_copy(
                    recs_hbm.at[c, s, 0], ibufs[_o], isems[_o]).wait()
                for q in range(CHUNK // 16):
                    colbuf[_o, pl.ds(q * 16, 16)] = (
                        ibufs[_o][CR, pl.ds(q * 16, 16)] & lomask)
                pltpu.async_copy(
                    emb_hbm.at[colbuf.at[_o]], gbufs[_o], gsems[_o])

            pltpu.make_async_copy(
                emb_hbm.at[colbuf.at[b]], gbufs[b], gsems[b]).wait()

            @plsc.parallel_loop(0, CHUNK // 16, unroll=1)
            def group_body(g, _b=b):
                vvec = lax.bitcast_convert_type(
                    ibufs[_b][VAL, pl.ds(g * 16, 16)], jnp.float32)
                for lane in range(16):
                    vv = jnp.full((16,), vvec[lane], dtype=jnp.float32)
                    e = g * 16 + lane
                    for d in range(D // 16):
                        sl = pl.ds(d * 16, 16)
                        sbuf[e, sl] = gbufs[_b][e, sl] * vv

            for q in range(CHUNK // 16):
                rowbuf[b, pl.ds(q * 16, 16)] = lax.shift_right_logical(
                    ibufs[b][CR, pl.ds(q * 16, 16)], 16)
            pltpu.sync_copy(sbuf, accum.at[rowbuf.at[b]], add=True)

            @pl.when(jj * 2 + b + 2 < NCH)
            def _(_b=b, _j=j):
                pltpu.async_copy(recs_hbm.at[c, s, _j + 2], ibufs[_b], isems[_b])
        return carry

    lax.fori_loop(0, npair, pair_body, 0, unroll=False)

    plsc.subcore_barrier()

    pltpu.sync_copy(
        accum.at[pl.ds(s * ROWS_PER_TILE, ROWS_PER_TILE)],
        out_hbm.at[c, s],
    )


@jax.jit
def kernel(users_emb, items_emb, user_indices, user_values, item_indices, item_values):
    emb = jnp.concatenate([users_emb, items_emb], axis=0)

    def prep(a):
        a = a.astype(jnp.int32)
        a = jnp.concatenate([a, jnp.zeros((E_PAD - E,), jnp.int32)])
        return a.reshape(NUM_SUBCORES, NCH, 1, CHUNK)

    def colrow(idx, off):
        return idx[1].astype(jnp.int32) + off + idx[0].astype(jnp.int32) * 65536

    recs = jnp.stack([
        jnp.concatenate([
            prep(colrow(user_indices, 0)),
            prep(lax.bitcast_convert_type(user_values, jnp.int32)),
        ], axis=2),
        jnp.concatenate([
            prep(colrow(item_indices, N_ROWS)),
            prep(lax.bitcast_convert_type(item_values, jnp.int32)),
        ], axis=2),
    ])
    zeros = jnp.zeros((ROWS_PER_TILE, D), jnp.float32)

    mesh = plsc.VectorSubcoreMesh(
        core_axis_name="c", subcore_axis_name="s",
        num_cores=NUM_CORES, num_subcores=NUM_SUBCORES,
    )
    out = pl.kernel(
        _spmm_body,
        out_type=jax.ShapeDtypeStruct(
            (NUM_CORES, NUM_SUBCORES, ROWS_PER_TILE, D), jnp.float32),
        mesh=mesh,
        compiler_params=pltpu.CompilerParams(use_tc_tiling_on_sc=False),
        scratch_types=[
            pltpu.VMEM((2, CHUNK), jnp.int32),
            pltpu.VMEM((2, CHUNK), jnp.int32),
            pltpu.VMEM((CHUNK, D), jnp.float32),
            pltpu.VMEM((CHUNK, D), jnp.float32),
            pltpu.VMEM((CHUNK, D), jnp.float32),
            pltpu.VMEM((2, CHUNK), jnp.int32),
            pltpu.VMEM((2, CHUNK), jnp.int32),
            pltpu.VMEM_SHARED((N_ROWS, D), jnp.float32),
            pltpu.SemaphoreType.DMA,
            pltpu.SemaphoreType.DMA,
            pltpu.SemaphoreType.DMA,
            pltpu.SemaphoreType.DMA,
        ],
    )(emb, recs, zeros)

    out = out.reshape(NUM_CORES, N_ROWS, D)
    return (out[0], out[1])
